# baseline (speedup 1.0000x reference)
import jax
import jax.numpy as jnp
from jax import lax
from jax.experimental import pallas as pl
from jax.experimental.pallas import tpu as pltpu

B, S, D = 16, 4096, 1024
BS = 512
HB = BS // 2
NBLK = S // BS
MAXT = B * NBLK


def _tc_body(lens_ref, batch_tbl, blk_tbl, x0_ref, x1_ref, out_ref, acc_ref):
    t = pl.program_id(0)
    i = batch_tbl[t]
    j = blk_tbl[t]
    length = lens_ref[i]
    last = lax.div(length - 1, BS)

    @pl.when(j == 0)
    def _init():
        acc_ref[...] = jnp.zeros_like(acc_ref)

    @pl.when(j < last)
    def _acc_full():
        acc_ref[...] += jnp.sum(
            x0_ref[0].reshape(HB // 8, 8, D), axis=0
        ) + jnp.sum(x1_ref[0].reshape(HB // 8, 8, D), axis=0)

    @pl.when(j == last)
    def _acc_tail():
        r0 = jax.lax.broadcasted_iota(jnp.int32, (HB, 1), 0) + j * BS
        m0 = jnp.where(r0 < length, x0_ref[0], 0.0)
        m1 = jnp.where(r0 + HB < length, x1_ref[0], 0.0)
        acc = (
            acc_ref[...]
            + jnp.sum(m0.reshape(HB // 8, 8, D), axis=0)
            + jnp.sum(m1.reshape(HB // 8, 8, D), axis=0)
        )
        total = jnp.sum(acc, axis=0, keepdims=True)
        out_ref[...] = (total * (1.0 / length.astype(jnp.float32)))[None]


@jax.jit
def kernel(input, length):
    lens = length.astype(jnp.int32)
    nb = (lens + (BS - 1)) // BS
    ends = jnp.cumsum(nb)
    starts = ends - nb
    total = ends[-1]
    t_iota = jnp.arange(MAXT, dtype=jnp.int32)
    batch_tbl = jnp.sum(
        (t_iota[:, None] >= ends[None, :]).astype(jnp.int32), axis=1
    )
    batch_tbl = jnp.minimum(batch_tbl, B - 1)
    blk_tbl = t_iota - starts[batch_tbl]
    blk_tbl = jnp.clip(blk_tbl, 0, NBLK - 1)

    def x0_map(t, lens_ref, batch_tbl_ref, blk_tbl_ref):
        return (batch_tbl_ref[t], 2 * blk_tbl_ref[t], 0)

    def x1_map(t, lens_ref, batch_tbl_ref, blk_tbl_ref):
        return (batch_tbl_ref[t], 2 * blk_tbl_ref[t] + 1, 0)

    def out_map(t, lens_ref, batch_tbl_ref, blk_tbl_ref):
        return (batch_tbl_ref[t], 0, 0)

    grid_spec = pltpu.PrefetchScalarGridSpec(
        num_scalar_prefetch=3,
        grid=(total,),
        in_specs=[
            pl.BlockSpec((1, HB, D), x0_map),
            pl.BlockSpec((1, HB, D), x1_map),
        ],
        out_specs=pl.BlockSpec((1, 1, D), out_map),
        scratch_shapes=[pltpu.VMEM((8, D), jnp.float32)],
    )
    out = pl.pallas_call(
        _tc_body,
        grid_spec=grid_spec,
        out_shape=jax.ShapeDtypeStruct((B, 1, D), jnp.float32),
        compiler_params=pltpu.CompilerParams(
            dimension_semantics=("arbitrary",)
        ),
    )(lens, batch_tbl, blk_tbl, input, input)
    return out.reshape(B, D)


# speedup vs baseline: 1.2191x; 1.2191x over previous
import jax
import jax.numpy as jnp
from jax import lax
from jax.experimental import pallas as pl
from jax.experimental.pallas import tpu as pltpu

B, S, D = 16, 4096, 1024
CR = 256
NCH = S // CR
MAXT = B * NCH
NBUF = 4


def _tc_body(lens_ref, batch_tbl, blk_tbl, total_ref, x_ref, lensf_ref, out_ref,
             bufs, acc, sems):
    T = total_ref[0]

    def issue(t, slot):
        i = batch_tbl[t]
        j = blk_tbl[t]
        pltpu.make_async_copy(
            x_ref.at[i, pl.ds(j * CR, CR)], bufs.at[slot], sems.at[slot]
        ).start()

    def wait(slot):
        pltpu.make_async_copy(
            x_ref.at[0, pl.ds(0, CR)], bufs.at[slot], sems.at[slot]
        ).wait()

    acc[...] = jnp.zeros_like(acc)

    for slot in range(NBUF):

        @pl.when(slot < T)
        def _prime(slot=slot):
            issue(slot, slot)

    def round_body(rr, _):
        for slot in range(NBUF):
            t = rr * NBUF + slot

            @pl.when(t < T)
            def _step(t=t, slot=slot):
                wait(slot)
                i = batch_tbl[t]
                j = blk_tbl[t]
                length = lens_ref[i]
                row_ids = (
                    jax.lax.broadcasted_iota(jnp.int32, (CR, 1), 0) + j * CR
                )
                masked = jnp.where(row_ids < length, bufs[slot], 0.0)
                acc[i] += jnp.sum(masked.reshape(CR // 8, 8, D), axis=0)

                @pl.when(t + NBUF < T)
                def _next():
                    issue(t + NBUF, slot)

        return 0

    nrounds = lax.div(T + (NBUF - 1), NBUF)
    lax.fori_loop(0, nrounds, round_body, 0)

    total = jnp.sum(acc[...], axis=1)
    out_ref[...] = total / lensf_ref[:, 0:1]


@jax.jit
def kernel(input, length):
    lens = length.astype(jnp.int32)
    nb = (lens + (CR - 1)) // CR
    ends = jnp.cumsum(nb)
    starts = ends - nb
    total = ends[-1:]
    t_iota = jnp.arange(MAXT, dtype=jnp.int32)
    batch_tbl = jnp.sum(
        (t_iota[:, None] >= ends[None, :]).astype(jnp.int32), axis=1
    )
    batch_tbl = jnp.minimum(batch_tbl, B - 1)
    blk_tbl = t_iota - starts[batch_tbl]
    blk_tbl = jnp.clip(blk_tbl, 0, NCH - 1)
    lens_f = jnp.broadcast_to(lens.astype(jnp.float32)[:, None], (B, 128))

    grid_spec = pltpu.PrefetchScalarGridSpec(
        num_scalar_prefetch=4,
        grid=(1,),
        in_specs=[
            pl.BlockSpec(memory_space=pltpu.HBM),
            pl.BlockSpec((B, 128), lambda t, *_: (0, 0)),
        ],
        out_specs=pl.BlockSpec((B, D), lambda t, *_: (0, 0)),
        scratch_shapes=[
            pltpu.VMEM((NBUF, CR, D), jnp.float32),
            pltpu.VMEM((B, 8, D), jnp.float32),
            pltpu.SemaphoreType.DMA((NBUF,)),
        ],
    )
    return pl.pallas_call(
        _tc_body,
        grid_spec=grid_spec,
        out_shape=jax.ShapeDtypeStruct((B, D), jnp.float32),
        compiler_params=pltpu.CompilerParams(
            dimension_semantics=("arbitrary",)
        ),
    )(lens, batch_tbl, blk_tbl, total, input, lens_f)


# speedup vs baseline: 1.4078x; 1.1548x over previous
import jax
import jax.numpy as jnp
from jax import lax
from jax.experimental import pallas as pl
from jax.experimental.pallas import tpu as pltpu

B, S, D = 16, 4096, 1024
CR = 256
NCH = S // CR
MAXT = B * NCH
NBUF = 8


def _tc_body(lens_ref, batch_tbl, blk_tbl, total_ref, x_ref, lensf_ref, out_ref,
             bufs, acc, sems):
    T = total_ref[0]

    def issue(t, slot):
        i = batch_tbl[t]
        j = blk_tbl[t]
        pltpu.make_async_copy(
            x_ref.at[i, pl.ds(j * CR, CR)], bufs.at[slot], sems.at[slot]
        ).start()

    def wait(slot):
        pltpu.make_async_copy(
            x_ref.at[0, pl.ds(0, CR)], bufs.at[slot], sems.at[slot]
        ).wait()

    acc[...] = jnp.zeros_like(acc)

    for slot in range(NBUF):

        @pl.when(slot < T)
        def _prime(slot=slot):
            issue(slot, slot)

    def round_body(rr, _):
        for slot in range(NBUF):
            t = rr * NBUF + slot

            @pl.when(t < T)
            def _step(t=t, slot=slot):
                wait(slot)
                i = batch_tbl[t]
                j = blk_tbl[t]
                length = lens_ref[i]
                last = lax.div(length - 1, CR)

                @pl.when(j < last)
                def _full():
                    acc[i] += jnp.sum(
                        bufs[slot].reshape(CR // 8, 8, D), axis=0
                    )

                @pl.when(j == last)
                def _tail():
                    row_ids = (
                        jax.lax.broadcasted_iota(jnp.int32, (CR, 1), 0)
                        + j * CR
                    )
                    masked = jnp.where(row_ids < length, bufs[slot], 0.0)
                    acc[i] += jnp.sum(masked.reshape(CR // 8, 8, D), axis=0)

                @pl.when(t + NBUF < T)
                def _next():
                    issue(t + NBUF, slot)

        return 0

    nrounds = lax.div(T + (NBUF - 1), NBUF)
    lax.fori_loop(0, nrounds, round_body, 0)

    total = jnp.sum(acc[...], axis=1)
    out_ref[...] = total / lensf_ref[:, 0:1]


@jax.jit
def kernel(input, length):
    lens = length.astype(jnp.int32)
    nb = (lens + (CR - 1)) // CR
    ends = jnp.cumsum(nb)
    starts = ends - nb
    total = ends[-1:]
    t_iota = jnp.arange(MAXT, dtype=jnp.int32)
    batch_tbl = jnp.sum(
        (t_iota[:, None] >= ends[None, :]).astype(jnp.int32), axis=1
    )
    batch_tbl = jnp.minimum(batch_tbl, B - 1)
    blk_tbl = t_iota - starts[batch_tbl]
    blk_tbl = jnp.clip(blk_tbl, 0, NCH - 1)
    lens_f = jnp.broadcast_to(lens.astype(jnp.float32)[:, None], (B, 128))

    grid_spec = pltpu.PrefetchScalarGridSpec(
        num_scalar_prefetch=4,
        grid=(1,),
        in_specs=[
            pl.BlockSpec(memory_space=pltpu.HBM),
            pl.BlockSpec((B, 128), lambda t, *_: (0, 0)),
        ],
        out_specs=pl.BlockSpec((B, D), lambda t, *_: (0, 0)),
        scratch_shapes=[
            pltpu.VMEM((NBUF, CR, D), jnp.float32),
            pltpu.VMEM((B, 8, D), jnp.float32),
            pltpu.SemaphoreType.DMA((NBUF,)),
        ],
    )
    return pl.pallas_call(
        _tc_body,
        grid_spec=grid_spec,
        out_shape=jax.ShapeDtypeStruct((B, D), jnp.float32),
        compiler_params=pltpu.CompilerParams(
            dimension_semantics=("arbitrary",)
        ),
    )(lens, batch_tbl, blk_tbl, total, input, lens_f)
